# baseline (speedup 1.0000x reference)
import functools

import jax
import jax.numpy as jnp
from jax import lax
from jax.experimental import pallas as pl
from jax.experimental.pallas import tpu as pltpu
from jax.experimental.pallas import tpu_sc as plsc

_B, _D, _P, _L, _K = 1024, 128, 100000, 32, 8
_TEMP = 0.1
_PBLK = 4096
_NSTEPS = (_P + _PBLK - 1) // _PBLK

_NC, _NS = 2, 16
_NW = _NC * _NS
_KW = _B * _K // _NW
_QW = _B // _NW


_LN = 128
_NCH = _PBLK // _LN


def _ce(a, b):
    av, ai = a
    bv, bi = b
    p = av >= bv
    return ((jnp.where(p, av, bv), jnp.where(p, ai, bi)),
            (jnp.where(p, bv, av), jnp.where(p, bi, ai)))


def _bitonic_sort_desc(xs):
    n = len(xs)
    if n == 1:
        return xs
    h = n // 2
    top, bot = [], []
    for i in range(h):
        hi_, lo = _ce(xs[i], xs[i + h])
        top.append(hi_)
        bot.append(lo)
    return _bitonic_sort_desc(top) + _bitonic_sort_desc(bot)


def _merge_top8(a, b):
    xs = a + b[::-1]
    top = [_ce(xs[i], xs[i + 8])[0] for i in range(8)]
    return _bitonic_sort_desc(top)


def _topk_body(q_ref, bank_ref, w_ref, idx_ref, stats_ref, vals_s, idx_s):
    step = pl.program_id(0)

    @pl.when(step == 0)
    def _init():
        vals_s[...] = jnp.full((_B, _K * _LN), -jnp.inf, jnp.float32)
        idx_s[...] = jnp.zeros((_B, _K * _LN), jnp.int32)

    q = q_ref[...]
    qn = q / jnp.maximum(jnp.sqrt(jnp.sum(q * q, axis=1, keepdims=True)), 1e-12)
    bank = bank_ref[...]
    bn = bank / jnp.maximum(
        jnp.sqrt(jnp.sum(bank * bank, axis=1, keepdims=True)), 1e-12)
    sims = lax.dot_general(qn, bn, (((1,), (1,)), ((), ())),
                           preferred_element_type=jnp.float32)
    lane = lax.broadcasted_iota(jnp.int32, (_B, _LN), 1)
    base = step * _PBLK

    _first_oob_chunk = (_P % _PBLK) // _LN
    lists = []
    for c in range(_NCH):
        v = sims[:, c * _LN:(c + 1) * _LN]
        gc = lane + (base + c * _LN)
        if c >= _first_oob_chunk:
            v = jnp.where(gc < _P, v, -jnp.inf)
        lists.append([(v, gc)])
    while len(lists) > 1:
        nxt = []
        for i in range(0, len(lists), 2):
            a, b = lists[i], lists[i + 1]
            if len(a) == 8:
                nxt.append(_merge_top8(a, b))
            else:
                nxt.append(_bitonic_sort_desc(a + b[::-1]))
        lists = nxt
    blk8 = lists[0]

    carry = [(vals_s[:, r * _LN:(r + 1) * _LN], idx_s[:, r * _LN:(r + 1) * _LN])
             for r in range(_K)]
    merged = _merge_top8(carry, blk8)
    for r in range(_K):
        vals_s[:, r * _LN:(r + 1) * _LN] = merged[r][0]
        idx_s[:, r * _LN:(r + 1) * _LN] = merged[r][1]

    @pl.when(step == _NSTEPS - 1)
    def _emit():
        cv = vals_s[...]
        ci = idx_s[...]
        nv, ni = [], []
        for _ in range(_K):
            m = jnp.max(cv, axis=1, keepdims=True)
            am = jnp.min(jnp.where(cv == m, ci, (1 << 30)), axis=1,
                         keepdims=True)
            nv.append(m)
            ni.append(am)
            cv = jnp.where(ci == am, -jnp.inf, cv)
        v = jnp.concatenate(nv, axis=1)
        mx = v[:, 0:1]
        e = jnp.exp((v - mx) / _TEMP)
        w_ref[...] = e / jnp.sum(e, axis=1, keepdims=True)
        idx_ref[...] = jnp.concatenate(ni, axis=1)
        stats_ref[...] = jnp.concatenate(
            [mx, jnp.mean(v, axis=1, keepdims=True)], axis=1)


def _run_topk(embeddings, prototype_vectors):
    return pl.pallas_call(
        _topk_body,
        grid=(_NSTEPS,),
        in_specs=[
            pl.BlockSpec((_B, _D), lambda i: (0, 0)),
            pl.BlockSpec((_PBLK, _D), lambda i: (i, 0)),
        ],
        out_specs=[
            pl.BlockSpec((_B, _K), lambda i: (0, 0)),
            pl.BlockSpec((_B, _K), lambda i: (0, 0)),
            pl.BlockSpec((_B, 2), lambda i: (0, 0)),
        ],
        out_shape=[
            jax.ShapeDtypeStruct((_B, _K), jnp.float32),
            jax.ShapeDtypeStruct((_B, _K), jnp.int32),
            jax.ShapeDtypeStruct((_B, 2), jnp.float32),
        ],
        scratch_shapes=[
            pltpu.VMEM((_B, _K * _LN), jnp.float32),
            pltpu.VMEM((_B, _K * _LN), jnp.int32),
        ],
    )(embeddings, prototype_vectors)


def _sc_gather_body(idx_hbm, w_hbm, tab_hbm, out_hbm,
                    idx_v, w_v, rows_v, out_v, sem):
    cid = lax.axis_index("c")
    sid = lax.axis_index("s")
    wid = sid * _NC + cid
    pltpu.sync_copy(idx_hbm.at[pl.ds(wid * 2, 2)], idx_v)
    pltpu.sync_copy(w_hbm.at[pl.ds(wid * _KW, _KW)], w_v)
    cp0 = pltpu.async_copy(tab_hbm.at[idx_v.at[0]],
                           rows_v.at[pl.ds(0, 128)], sem)
    cp1 = pltpu.async_copy(tab_hbm.at[idx_v.at[1]],
                           rows_v.at[pl.ds(128, 128)], sem)
    cp0.wait()
    cp1.wait()
    for qi in range(_QW):
        for c in range(_L // 16):
            sl = pl.ds(c * 16, 16)
            acc = w_v[qi * _K, sl] * rows_v[qi * _K, sl]
            for k in range(1, _K):
                r = qi * _K + k
                acc = acc + w_v[r, sl] * rows_v[r, sl]
            out_v[qi, sl] = acc
    pltpu.sync_copy(out_v, out_hbm.at[pl.ds(wid * _QW, _QW)])


def _run_sc_gather(idx2, w_flat, soft_labels):
    mesh = plsc.VectorSubcoreMesh(core_axis_name="c", subcore_axis_name="s")
    f = functools.partial(
        pl.kernel,
        out_type=jax.ShapeDtypeStruct((_B, _L), jnp.float32),
        mesh=mesh,
        compiler_params=pltpu.CompilerParams(use_tc_tiling_on_sc=False),
        scratch_types=[
            pltpu.VMEM((2, 128), jnp.int32),
            pltpu.VMEM((_KW, _L), jnp.float32),
            pltpu.VMEM((_KW, _L), jnp.float32),
            pltpu.VMEM((_QW, _L), jnp.float32),
            pltpu.SemaphoreType.DMA,
        ],
    )(_sc_gather_body)
    return f(idx2, w_flat, soft_labels)


def kernel(embeddings, prototype_vectors, soft_labels, label_indices,
           state_values):
    weights, top_idx, stats = _run_topk(embeddings, prototype_vectors)
    idx2 = top_idx.reshape(_NW * 2, 128)
    wb = jnp.broadcast_to(weights[:, :, None], (_B, _K, _L)).reshape(
        _B * _K, _L)
    prior = _run_sc_gather(idx2, wb, soft_labels)
    return prior, stats


# speedup vs baseline: 1.1132x; 1.1132x over previous
import functools

import jax
import jax.numpy as jnp
from jax import lax
from jax.experimental import pallas as pl
from jax.experimental.pallas import tpu as pltpu
from jax.experimental.pallas import tpu_sc as plsc

_B, _D, _P, _L, _K = 1024, 128, 100000, 32, 8
_TEMP = 0.1
_PBLK = 2048
_NSTEPS = (_P + _PBLK - 1) // _PBLK

_NC, _NS = 2, 16
_NW = _NC * _NS
_KW = _B * _K // _NW
_QW = _B // _NW


_LN = 128
_NCH = _PBLK // _LN


def _ce(a, b):
    av, ai = a
    bv, bi = b
    p = av >= bv
    return ((jnp.where(p, av, bv), jnp.where(p, ai, bi)),
            (jnp.where(p, bv, av), jnp.where(p, bi, ai)))


def _bitonic_sort_desc(xs):
    n = len(xs)
    if n == 1:
        return xs
    h = n // 2
    top, bot = [], []
    for i in range(h):
        hi_, lo = _ce(xs[i], xs[i + h])
        top.append(hi_)
        bot.append(lo)
    return _bitonic_sort_desc(top) + _bitonic_sort_desc(bot)


def _merge_top8(a, b):
    xs = a + b[::-1]
    top = [_ce(xs[i], xs[i + 8])[0] for i in range(8)]
    return _bitonic_sort_desc(top)


def _merge2(a, b):
    h0, l0 = _ce(a[0], b[0])
    h1, l1 = _ce(a[1], b[1])
    m0, m1 = _ce(l0, h1)
    return [h0, m0, m1, l1]


def _merge4(a, b):
    e = _merge2([a[0], a[2]], [b[0], b[2]])
    o = _merge2([a[1], a[3]], [b[1], b[3]])
    out = [e[0]]
    for i in range(3):
        hi_, lo = _ce(o[i], e[i + 1])
        out += [hi_, lo]
    out.append(o[3])
    return out


def _topk_body(q_ref, bank_ref, w_ref, idx_ref, stats_ref, vals_s, idx_s):
    step = pl.program_id(0)

    @pl.when(step == 0)
    def _init():
        vals_s[...] = jnp.full((_B, _K * _LN), -jnp.inf, jnp.float32)
        idx_s[...] = jnp.zeros((_B, _K * _LN), jnp.int32)

    q = q_ref[...]
    qn = q / jnp.maximum(jnp.sqrt(jnp.sum(q * q, axis=1, keepdims=True)), 1e-12)
    bank = bank_ref[...]
    bn = bank / jnp.maximum(
        jnp.sqrt(jnp.sum(bank * bank, axis=1, keepdims=True)), 1e-12)
    sims = lax.dot_general(qn, bn, (((1,), (1,)), ((), ())),
                           preferred_element_type=jnp.float32)
    lane = lax.broadcasted_iota(jnp.int32, (_B, _LN), 1)
    base = step * _PBLK

    _first_oob_chunk = (_P % _PBLK) // _LN
    lists = []
    for c in range(_NCH):
        v = sims[:, c * _LN:(c + 1) * _LN]
        gc = lane + (base + c * _LN)
        if c >= _first_oob_chunk:
            v = jnp.where(gc < _P, v, -jnp.inf)
        lists.append([(v, gc)])
    while len(lists) > 1:
        nxt = []
        for i in range(0, len(lists), 2):
            a, b = lists[i], lists[i + 1]
            if len(a) == 1:
                hi_, lo = _ce(a[0], b[0])
                nxt.append([hi_, lo])
            elif len(a) == 2:
                nxt.append(_merge2(a, b))
            elif len(a) == 4:
                nxt.append(_merge4(a, b))
            else:
                nxt.append(_merge_top8(a, b))
        lists = nxt
    blk8 = lists[0]

    carry = [(vals_s[:, r * _LN:(r + 1) * _LN], idx_s[:, r * _LN:(r + 1) * _LN])
             for r in range(_K)]
    merged = _merge_top8(carry, blk8)
    for r in range(_K):
        vals_s[:, r * _LN:(r + 1) * _LN] = merged[r][0]
        idx_s[:, r * _LN:(r + 1) * _LN] = merged[r][1]

    @pl.when(step == _NSTEPS - 1)
    def _emit():
        cv = vals_s[...]
        ci = idx_s[...]
        nv, ni = [], []
        for _ in range(_K):
            m = jnp.max(cv, axis=1, keepdims=True)
            am = jnp.min(jnp.where(cv == m, ci, (1 << 30)), axis=1,
                         keepdims=True)
            nv.append(m)
            ni.append(am)
            cv = jnp.where(ci == am, -jnp.inf, cv)
        v = jnp.concatenate(nv, axis=1)
        mx = v[:, 0:1]
        e = jnp.exp((v - mx) / _TEMP)
        w_ref[...] = e / jnp.sum(e, axis=1, keepdims=True)
        idx_ref[...] = jnp.concatenate(ni, axis=1)
        stats_ref[...] = jnp.concatenate(
            [mx, jnp.mean(v, axis=1, keepdims=True)], axis=1)


def _run_topk(embeddings, prototype_vectors):
    return pl.pallas_call(
        _topk_body,
        grid=(_NSTEPS,),
        in_specs=[
            pl.BlockSpec((_B, _D), lambda i: (0, 0)),
            pl.BlockSpec((_PBLK, _D), lambda i: (i, 0)),
        ],
        out_specs=[
            pl.BlockSpec((_B, _K), lambda i: (0, 0)),
            pl.BlockSpec((_B, _K), lambda i: (0, 0)),
            pl.BlockSpec((_B, 2), lambda i: (0, 0)),
        ],
        out_shape=[
            jax.ShapeDtypeStruct((_B, _K), jnp.float32),
            jax.ShapeDtypeStruct((_B, _K), jnp.int32),
            jax.ShapeDtypeStruct((_B, 2), jnp.float32),
        ],
        scratch_shapes=[
            pltpu.VMEM((_B, _K * _LN), jnp.float32),
            pltpu.VMEM((_B, _K * _LN), jnp.int32),
        ],
    )(embeddings, prototype_vectors)


def _sc_gather_body(idx_hbm, w_hbm, tab_hbm, out_hbm,
                    idx_v, w_v, rows_v, out_v, sem):
    cid = lax.axis_index("c")
    sid = lax.axis_index("s")
    wid = sid * _NC + cid
    pltpu.sync_copy(idx_hbm.at[pl.ds(wid * 2, 2)], idx_v)
    pltpu.sync_copy(w_hbm.at[pl.ds(wid * _KW, _KW)], w_v)
    cp0 = pltpu.async_copy(tab_hbm.at[idx_v.at[0]],
                           rows_v.at[pl.ds(0, 128)], sem)
    cp1 = pltpu.async_copy(tab_hbm.at[idx_v.at[1]],
                           rows_v.at[pl.ds(128, 128)], sem)
    cp0.wait()
    cp1.wait()
    for qi in range(_QW):
        for c in range(_L // 16):
            sl = pl.ds(c * 16, 16)
            acc = w_v[qi * _K, sl] * rows_v[qi * _K, sl]
            for k in range(1, _K):
                r = qi * _K + k
                acc = acc + w_v[r, sl] * rows_v[r, sl]
            out_v[qi, sl] = acc
    pltpu.sync_copy(out_v, out_hbm.at[pl.ds(wid * _QW, _QW)])


def _run_sc_gather(idx2, w_flat, soft_labels):
    mesh = plsc.VectorSubcoreMesh(core_axis_name="c", subcore_axis_name="s")
    f = functools.partial(
        pl.kernel,
        out_type=jax.ShapeDtypeStruct((_B, _L), jnp.float32),
        mesh=mesh,
        compiler_params=pltpu.CompilerParams(use_tc_tiling_on_sc=False),
        scratch_types=[
            pltpu.VMEM((2, 128), jnp.int32),
            pltpu.VMEM((_KW, _L), jnp.float32),
            pltpu.VMEM((_KW, _L), jnp.float32),
            pltpu.VMEM((_QW, _L), jnp.float32),
            pltpu.SemaphoreType.DMA,
        ],
    )(_sc_gather_body)
    return f(idx2, w_flat, soft_labels)


def kernel(embeddings, prototype_vectors, soft_labels, label_indices,
           state_values):
    weights, top_idx, stats = _run_topk(embeddings, prototype_vectors)
    idx2 = top_idx.reshape(_NW * 2, 128)
    wb = jnp.broadcast_to(weights[:, :, None], (_B, _K, _L)).reshape(
        _B * _K, _L)
    prior = _run_sc_gather(idx2, wb, soft_labels)
    return prior, stats


# speedup vs baseline: 1.1144x; 1.0011x over previous
import functools

import jax
import jax.numpy as jnp
from jax import lax
from jax.experimental import pallas as pl
from jax.experimental.pallas import tpu as pltpu
from jax.experimental.pallas import tpu_sc as plsc

_B, _D, _P, _L, _K = 1024, 128, 100000, 32, 8
_TEMP = 0.1
_PBLK = 2048
_NSTEPS = (_P + _PBLK - 1) // _PBLK

_NC, _NS = 2, 16
_NW = _NC * _NS
_KW = _B * _K // _NW
_QW = _B // _NW


_LN = 128
_NCH = _PBLK // _LN


def _ce(a, b):
    av, ai = a
    bv, bi = b
    p = av >= bv
    return ((jnp.where(p, av, bv), jnp.where(p, ai, bi)),
            (jnp.where(p, bv, av), jnp.where(p, bi, ai)))


def _bitonic_sort_desc(xs):
    n = len(xs)
    if n == 1:
        return xs
    h = n // 2
    top, bot = [], []
    for i in range(h):
        hi_, lo = _ce(xs[i], xs[i + h])
        top.append(hi_)
        bot.append(lo)
    return _bitonic_sort_desc(top) + _bitonic_sort_desc(bot)


def _merge_top8(a, b):
    e = _merge4([a[0], a[2], a[4], a[6]], [b[0], b[2], b[4], b[6]])
    o = _merge4([a[1], a[3], a[5], a[7]], [b[1], b[3], b[5], b[7]])
    out = [e[0]]
    for i in range(3):
        hi_, lo = _ce(o[i], e[i + 1])
        out += [hi_, lo]
    out.append(_ce(o[3], e[4])[0])
    return out


def _merge2(a, b):
    h0, l0 = _ce(a[0], b[0])
    h1, l1 = _ce(a[1], b[1])
    m0, m1 = _ce(l0, h1)
    return [h0, m0, m1, l1]


def _merge4(a, b):
    e = _merge2([a[0], a[2]], [b[0], b[2]])
    o = _merge2([a[1], a[3]], [b[1], b[3]])
    out = [e[0]]
    for i in range(3):
        hi_, lo = _ce(o[i], e[i + 1])
        out += [hi_, lo]
    out.append(o[3])
    return out


def _topk_body(q_ref, bank_ref, w_ref, idx_ref, stats_ref, vals_s, idx_s):
    step = pl.program_id(0)

    @pl.when(step == 0)
    def _init():
        vals_s[...] = jnp.full((_B, _K * _LN), -jnp.inf, jnp.float32)
        idx_s[...] = jnp.zeros((_B, _K * _LN), jnp.int32)

    q = q_ref[...]
    qn = q / jnp.maximum(jnp.sqrt(jnp.sum(q * q, axis=1, keepdims=True)), 1e-12)
    bank = bank_ref[...]
    bn = bank / jnp.maximum(
        jnp.sqrt(jnp.sum(bank * bank, axis=1, keepdims=True)), 1e-12)
    sims = lax.dot_general(qn, bn, (((1,), (1,)), ((), ())),
                           preferred_element_type=jnp.float32)
    lane = lax.broadcasted_iota(jnp.int32, (_B, _LN), 1)
    base = step * _PBLK

    _first_oob_chunk = (_P % _PBLK) // _LN
    lists = []
    for c in range(_NCH):
        v = sims[:, c * _LN:(c + 1) * _LN]
        gc = lane + (base + c * _LN)
        if c >= _first_oob_chunk:
            v = jnp.where(gc < _P, v, -jnp.inf)
        lists.append([(v, gc)])
    while len(lists) > 1:
        nxt = []
        for i in range(0, len(lists), 2):
            a, b = lists[i], lists[i + 1]
            if len(a) == 1:
                hi_, lo = _ce(a[0], b[0])
                nxt.append([hi_, lo])
            elif len(a) == 2:
                nxt.append(_merge2(a, b))
            elif len(a) == 4:
                nxt.append(_merge4(a, b))
            else:
                nxt.append(_merge_top8(a, b))
        lists = nxt
    blk8 = lists[0]

    carry = [(vals_s[:, r * _LN:(r + 1) * _LN], idx_s[:, r * _LN:(r + 1) * _LN])
             for r in range(_K)]
    merged = _merge_top8(carry, blk8)
    for r in range(_K):
        vals_s[:, r * _LN:(r + 1) * _LN] = merged[r][0]
        idx_s[:, r * _LN:(r + 1) * _LN] = merged[r][1]

    @pl.when(step == _NSTEPS - 1)
    def _emit():
        cv = vals_s[...]
        ci = idx_s[...]
        nv, ni = [], []
        for _ in range(_K):
            m = jnp.max(cv, axis=1, keepdims=True)
            am = jnp.min(jnp.where(cv == m, ci, (1 << 30)), axis=1,
                         keepdims=True)
            nv.append(m)
            ni.append(am)
            cv = jnp.where(ci == am, -jnp.inf, cv)
        v = jnp.concatenate(nv, axis=1)
        mx = v[:, 0:1]
        e = jnp.exp((v - mx) / _TEMP)
        w_ref[...] = e / jnp.sum(e, axis=1, keepdims=True)
        idx_ref[...] = jnp.concatenate(ni, axis=1)
        stats_ref[...] = jnp.concatenate(
            [mx, jnp.mean(v, axis=1, keepdims=True)], axis=1)


def _run_topk(embeddings, prototype_vectors):
    return pl.pallas_call(
        _topk_body,
        grid=(_NSTEPS,),
        in_specs=[
            pl.BlockSpec((_B, _D), lambda i: (0, 0)),
            pl.BlockSpec((_PBLK, _D), lambda i: (i, 0)),
        ],
        out_specs=[
            pl.BlockSpec((_B, _K), lambda i: (0, 0)),
            pl.BlockSpec((_B, _K), lambda i: (0, 0)),
            pl.BlockSpec((_B, 2), lambda i: (0, 0)),
        ],
        out_shape=[
            jax.ShapeDtypeStruct((_B, _K), jnp.float32),
            jax.ShapeDtypeStruct((_B, _K), jnp.int32),
            jax.ShapeDtypeStruct((_B, 2), jnp.float32),
        ],
        scratch_shapes=[
            pltpu.VMEM((_B, _K * _LN), jnp.float32),
            pltpu.VMEM((_B, _K * _LN), jnp.int32),
        ],
    )(embeddings, prototype_vectors)


def _sc_gather_body(idx_hbm, w_hbm, tab_hbm, out_hbm,
                    idx_v, w_v, rows_v, out_v, sem):
    cid = lax.axis_index("c")
    sid = lax.axis_index("s")
    wid = sid * _NC + cid
    pltpu.sync_copy(idx_hbm.at[pl.ds(wid * 2, 2)], idx_v)
    pltpu.sync_copy(w_hbm.at[pl.ds(wid * _KW, _KW)], w_v)
    cp0 = pltpu.async_copy(tab_hbm.at[idx_v.at[0]],
                           rows_v.at[pl.ds(0, 128)], sem)
    cp1 = pltpu.async_copy(tab_hbm.at[idx_v.at[1]],
                           rows_v.at[pl.ds(128, 128)], sem)
    cp0.wait()
    cp1.wait()
    for qi in range(_QW):
        for c in range(_L // 16):
            sl = pl.ds(c * 16, 16)
            acc = w_v[qi * _K, sl] * rows_v[qi * _K, sl]
            for k in range(1, _K):
                r = qi * _K + k
                acc = acc + w_v[r, sl] * rows_v[r, sl]
            out_v[qi, sl] = acc
    pltpu.sync_copy(out_v, out_hbm.at[pl.ds(wid * _QW, _QW)])


def _run_sc_gather(idx2, w_flat, soft_labels):
    mesh = plsc.VectorSubcoreMesh(core_axis_name="c", subcore_axis_name="s")
    f = functools.partial(
        pl.kernel,
        out_type=jax.ShapeDtypeStruct((_B, _L), jnp.float32),
        mesh=mesh,
        compiler_params=pltpu.CompilerParams(use_tc_tiling_on_sc=False),
        scratch_types=[
            pltpu.VMEM((2, 128), jnp.int32),
            pltpu.VMEM((_KW, _L), jnp.float32),
            pltpu.VMEM((_KW, _L), jnp.float32),
            pltpu.VMEM((_QW, _L), jnp.float32),
            pltpu.SemaphoreType.DMA,
        ],
    )(_sc_gather_body)
    return f(idx2, w_flat, soft_labels)


def kernel(embeddings, prototype_vectors, soft_labels, label_indices,
           state_values):
    weights, top_idx, stats = _run_topk(embeddings, prototype_vectors)
    idx2 = top_idx.reshape(_NW * 2, 128)
    wb = jnp.broadcast_to(weights[:, :, None], (_B, _K, _L)).reshape(
        _B * _K, _L)
    prior = _run_sc_gather(idx2, wb, soft_labels)
    return prior, stats
